# baseline (speedup 1.0000x reference)
import functools

import jax
import jax.numpy as jnp
from jax import lax
from jax.experimental import pallas as pl
from jax.experimental.pallas import tpu as pltpu
from jax.experimental.pallas import tpu_sc as plsc

F = 26
V = 100000
K = 16
B = 4096
BF = B * F
FV = F * V
NBLK = FV // 8

NC = 2
NS = 16
NW = NC * NS
RPW = BF // NW
CH = 128
NCH = RPW // CH
NGRP = CH // 16


def _sc_emb_body(idx_hbm, emb_hbm, out_hbm, idx_v, idxb_v, blk0, blk1,
                 out_v, sem0, sem1):
    wid = lax.axis_index("s") * NC + lax.axis_index("c")
    base = pl.multiple_of(wid * RPW, RPW)
    pltpu.sync_copy(idx_hbm.at[pl.ds(base, RPW)], idx_v)

    def mkblk(i, carry):
        off = pl.multiple_of(i * 16, 16)
        idxb_v[pl.ds(off, 16)] = lax.shift_right_logical(
            idx_v[pl.ds(off, 16)], 3)
        return carry
    lax.fori_loop(0, RPW // 16, mkblk, 0)

    def fire(c, blk, sem):
        off = pl.multiple_of(c * CH, CH)
        pltpu.async_copy(emb_hbm.at[idxb_v.at[pl.ds(off, CH)]], blk, sem)

    def wait(c, blk, sem):
        off = pl.multiple_of(c * CH, CH)
        pltpu.make_async_copy(
            emb_hbm.at[idxb_v.at[pl.ds(off, CH)]], blk, sem).wait()

    lanes = lax.iota(jnp.int32, 16)

    def extract(c, blk):
        def group(g, carry):
            off = pl.multiple_of(c * CH + g * 16, 16)
            gidx = idx_v[pl.ds(off, 16)]
            colb = lax.shift_left(jnp.bitwise_and(gidx, 7), 4)
            local_r = lanes + g * 16
            jv = lanes + off
            orow = lax.shift_right_logical(jv, 3)
            ocolb = lax.shift_left(jnp.bitwise_and(jv, 7), 4)
            for k in range(K):
                vals = plsc.load_gather(blk, [local_r, colb + k])
                plsc.store_scatter(out_v, [orow, ocolb + k], vals)
            return carry
        lax.fori_loop(0, NGRP, group, 0)

    fire(0, blk0, sem0)
    fire(1, blk1, sem1)

    def step(i, carry):
        c0 = i * 2
        c1 = i * 2 + 1
        wait(c0, blk0, sem0)
        extract(c0, blk0)

        @pl.when(c0 + 2 < NCH)
        def _():
            fire(c0 + 2, blk0, sem0)

        wait(c1, blk1, sem1)
        extract(c1, blk1)

        @pl.when(c1 + 2 < NCH)
        def _():
            fire(c1 + 2, blk1, sem1)
        return carry

    lax.fori_loop(0, NCH // 2, step, 0)

    obase = pl.multiple_of(wid * (RPW // 8), RPW // 8)
    pltpu.sync_copy(out_v, out_hbm.at[pl.ds(obase, RPW // 8)])


@functools.cache
def _sc_emb():
    return pl.kernel(
        _sc_emb_body,
        out_type=jax.ShapeDtypeStruct((BF // 8, 128), jnp.float32),
        mesh=plsc.VectorSubcoreMesh(core_axis_name="c", subcore_axis_name="s"),
        scratch_types=[
            pltpu.VMEM((RPW,), jnp.int32),
            pltpu.VMEM((RPW,), jnp.int32),
            pltpu.VMEM((CH, 128), jnp.float32),
            pltpu.VMEM((CH, 128), jnp.float32),
            pltpu.VMEM((RPW // 8, 128), jnp.float32),
            pltpu.SemaphoreType.DMA,
            pltpu.SemaphoreType.DMA,
        ],
        compiler_params=pltpu.CompilerParams(needs_layout_passes=False),
    )


def _sc_lin_body(idx_hbm, lin_hbm, lin_out, idx_v, lin_v, sem):
    wid = lax.axis_index("s") * NC + lax.axis_index("c")
    base = pl.multiple_of(wid * RPW, RPW)
    pltpu.sync_copy(idx_hbm.at[pl.ds(base, RPW)], idx_v)

    def fire(c, carry):
        off = pl.multiple_of(c * CH, CH)
        pltpu.async_copy(lin_hbm.at[idx_v.at[pl.ds(off, CH)]],
                         lin_v.at[pl.ds(off, CH)], sem)
        return carry
    lax.fori_loop(0, NCH, fire, 0)

    def drain(c, carry):
        off = pl.multiple_of(c * CH, CH)
        pltpu.make_async_copy(lin_hbm.at[idx_v.at[pl.ds(off, CH)]],
                              lin_v.at[pl.ds(off, CH)], sem).wait()
        return carry
    lax.fori_loop(0, NCH, drain, 0)

    pltpu.sync_copy(lin_v, lin_out.at[pl.ds(base, RPW)])


@functools.cache
def _sc_lin():
    return pl.kernel(
        _sc_lin_body,
        out_type=jax.ShapeDtypeStruct((BF,), jnp.float32),
        mesh=plsc.VectorSubcoreMesh(core_axis_name="c", subcore_axis_name="s"),
        scratch_types=[
            pltpu.VMEM((RPW,), jnp.int32),
            pltpu.VMEM((RPW,), jnp.float32),
            pltpu.SemaphoreType.DMA,
        ],
        compiler_params=pltpu.CompilerParams(use_tc_tiling_on_sc=False),
    )


def _tc_dense_body(flat_ref, lin_ref, linb_ref, w1_ref, b1_ref, w2_ref,
                   b2_ref, w3_ref, b3_ref, out_ref):
    x = flat_ref[...]
    s = x[:, 0:K]
    for f in range(1, F):
        s = s + x[:, f * K:(f + 1) * K]
    sum_of_square = jnp.sum(s * s)
    square_of_sum = jnp.sum(x * x)
    interaction = 0.5 * (sum_of_square - square_of_sum)
    lin = lin_ref[...]
    line_out = jnp.sum(lin, axis=1, keepdims=True) + linb_ref[...]
    h = jnp.dot(x, w1_ref[...], preferred_element_type=jnp.float32)
    h = jnp.maximum(h + b1_ref[...], 0.0)
    h = jnp.dot(h, w2_ref[...], preferred_element_type=jnp.float32)
    h = jnp.maximum(h + b2_ref[...], 0.0)
    fnn = jnp.dot(h, w3_ref[...], preferred_element_type=jnp.float32)
    fnn = fnn + b3_ref[...]
    out_ref[...] = fnn + line_out + interaction


_tc_dense = pl.pallas_call(
    _tc_dense_body,
    out_shape=jax.ShapeDtypeStruct((B, 1), jnp.float32),
)


def kernel(inputs, emb_table, lin_table, lin_bias, W1, b1, W2, b2, W3, b3):
    flat_idx = (inputs + jnp.arange(F, dtype=jnp.int32)[None, :] * V)
    flat_idx = flat_idx.reshape(BF)
    emb_blocks = emb_table.reshape(NBLK, 128)
    lin_flat = lin_table.reshape(FV)
    emb_rows = _sc_emb()(flat_idx, emb_blocks)
    lin_vals = _sc_lin()(flat_idx, lin_flat)
    flat = emb_rows.reshape(B, F * K)
    lin2 = lin_vals.reshape(B, F)
    return _tc_dense(flat, lin2, lin_bias, W1, b1, W2, b2, W3, b3)


# speedup vs baseline: 1.0005x; 1.0005x over previous
import functools

import jax
import jax.numpy as jnp
from jax import lax
from jax.experimental import pallas as pl
from jax.experimental.pallas import tpu as pltpu
from jax.experimental.pallas import tpu_sc as plsc

F = 26
V = 100000
K = 16
B = 4096
BF = B * F
FV = F * V
NBLK = FV // 8

NC = 2
NS = 16
NW = NC * NS
RPW = BF // NW
CH = 128
NCH = RPW // CH
NGRP = CH // 16


def _sc_emb_body(idx_hbm, emb_hbm, out_hbm, idx_v, idxb_v, blk0, blk1,
                 out_v, sem0, sem1):
    wid = lax.axis_index("s") * NC + lax.axis_index("c")
    base = pl.multiple_of(wid * RPW, RPW)
    pltpu.sync_copy(idx_hbm.at[pl.ds(base, RPW)], idx_v)

    def mkblk(i, carry):
        off = pl.multiple_of(i * 16, 16)
        idxb_v[pl.ds(off, 16)] = lax.shift_right_logical(
            idx_v[pl.ds(off, 16)], 3)
        return carry
    lax.fori_loop(0, RPW // 16, mkblk, 0)

    def fire(c, blk, sem):
        off = pl.multiple_of(c * CH, CH)
        pltpu.async_copy(emb_hbm.at[idxb_v.at[pl.ds(off, CH)]], blk, sem)

    def wait(c, blk, sem):
        off = pl.multiple_of(c * CH, CH)
        pltpu.make_async_copy(
            emb_hbm.at[idxb_v.at[pl.ds(off, CH)]], blk, sem).wait()

    lanes = lax.iota(jnp.int32, 16)

    def extract(c, blk):
        def group(g, carry):
            off = pl.multiple_of(c * CH + g * 16, 16)
            gidx = idx_v[pl.ds(off, 16)]
            colb = lax.shift_left(jnp.bitwise_and(gidx, 7), 4)
            local_r = lanes + g * 16
            jv = lanes + off
            orow = lax.shift_right_logical(jv, 3)
            ocolb = lax.shift_left(jnp.bitwise_and(jv, 7), 4)
            for k in range(K):
                vals = plsc.load_gather(blk, [local_r, colb + k])
                plsc.store_scatter(out_v, [orow, ocolb + k], vals)
            return carry
        lax.fori_loop(0, NGRP, group, 0)

    fire(0, blk0, sem0)
    fire(1, blk1, sem1)

    def step(i, carry):
        c0 = i * 2
        c1 = i * 2 + 1
        wait(c0, blk0, sem0)
        extract(c0, blk0)

        @pl.when(c0 + 2 < NCH)
        def _():
            fire(c0 + 2, blk0, sem0)

        wait(c1, blk1, sem1)
        extract(c1, blk1)

        @pl.when(c1 + 2 < NCH)
        def _():
            fire(c1 + 2, blk1, sem1)
        return carry

    lax.fori_loop(0, NCH // 2, step, 0)

    obase = pl.multiple_of(wid * (RPW // 8), RPW // 8)
    pltpu.sync_copy(out_v, out_hbm.at[pl.ds(obase, RPW // 8)])


@functools.cache
def _sc_emb():
    return pl.kernel(
        _sc_emb_body,
        out_type=jax.ShapeDtypeStruct((BF // 8, 128), jnp.float32),
        mesh=plsc.VectorSubcoreMesh(core_axis_name="c", subcore_axis_name="s"),
        scratch_types=[
            pltpu.VMEM((RPW,), jnp.int32),
            pltpu.VMEM((RPW,), jnp.int32),
            pltpu.VMEM((CH, 128), jnp.float32),
            pltpu.VMEM((CH, 128), jnp.float32),
            pltpu.VMEM((RPW // 8, 128), jnp.float32),
            pltpu.SemaphoreType.DMA,
            pltpu.SemaphoreType.DMA,
        ],
        compiler_params=pltpu.CompilerParams(
            needs_layout_passes=False, use_tc_tiling_on_sc=True),
    )


def _sc_lin_body(idx_hbm, lin_hbm, lin_out, idx_v, lin_v, sem):
    wid = lax.axis_index("s") * NC + lax.axis_index("c")
    base = pl.multiple_of(wid * RPW, RPW)
    pltpu.sync_copy(idx_hbm.at[pl.ds(base, RPW)], idx_v)

    def fire(c, carry):
        off = pl.multiple_of(c * CH, CH)
        pltpu.async_copy(lin_hbm.at[idx_v.at[pl.ds(off, CH)]],
                         lin_v.at[pl.ds(off, CH)], sem)
        return carry
    lax.fori_loop(0, NCH, fire, 0)

    def drain(c, carry):
        off = pl.multiple_of(c * CH, CH)
        pltpu.make_async_copy(lin_hbm.at[idx_v.at[pl.ds(off, CH)]],
                              lin_v.at[pl.ds(off, CH)], sem).wait()
        return carry
    lax.fori_loop(0, NCH, drain, 0)

    pltpu.sync_copy(lin_v, lin_out.at[pl.ds(base, RPW)])


@functools.cache
def _sc_lin():
    return pl.kernel(
        _sc_lin_body,
        out_type=jax.ShapeDtypeStruct((BF,), jnp.float32),
        mesh=plsc.VectorSubcoreMesh(core_axis_name="c", subcore_axis_name="s"),
        scratch_types=[
            pltpu.VMEM((RPW,), jnp.int32),
            pltpu.VMEM((RPW,), jnp.float32),
            pltpu.SemaphoreType.DMA,
        ],
        compiler_params=pltpu.CompilerParams(use_tc_tiling_on_sc=False),
    )


def _tc_dense_body(flat_ref, lin_ref, linb_ref, w1_ref, b1_ref, w2_ref,
                   b2_ref, w3_ref, b3_ref, out_ref):
    x = flat_ref[...]
    s = x[:, 0:K]
    for f in range(1, F):
        s = s + x[:, f * K:(f + 1) * K]
    sum_of_square = jnp.sum(s * s)
    square_of_sum = jnp.sum(x * x)
    interaction = 0.5 * (sum_of_square - square_of_sum)
    lin = lin_ref[...]
    line_out = jnp.sum(lin, axis=1, keepdims=True) + linb_ref[...]
    h = jnp.dot(x, w1_ref[...], preferred_element_type=jnp.float32)
    h = jnp.maximum(h + b1_ref[...], 0.0)
    h = jnp.dot(h, w2_ref[...], preferred_element_type=jnp.float32)
    h = jnp.maximum(h + b2_ref[...], 0.0)
    fnn = jnp.dot(h, w3_ref[...], preferred_element_type=jnp.float32)
    fnn = fnn + b3_ref[...]
    out_ref[...] = fnn + line_out + interaction


_tc_dense = pl.pallas_call(
    _tc_dense_body,
    out_shape=jax.ShapeDtypeStruct((B, 1), jnp.float32),
)


def kernel(inputs, emb_table, lin_table, lin_bias, W1, b1, W2, b2, W3, b3):
    flat_idx = (inputs + jnp.arange(F, dtype=jnp.int32)[None, :] * V)
    flat_idx = flat_idx.reshape(BF)
    emb_blocks = emb_table.reshape(NBLK, 128)
    lin_flat = lin_table.reshape(FV)
    emb_rows = _sc_emb()(flat_idx, emb_blocks)
    lin_vals = _sc_lin()(flat_idx, lin_flat)
    flat = emb_rows.reshape(B, F * K)
    lin2 = lin_vals.reshape(B, F)
    return _tc_dense(flat, lin2, lin_bias, W1, b1, W2, b2, W3, b3)
